# baseline (speedup 1.0000x reference)
import functools

import jax
import jax.numpy as jnp
from jax import lax
from jax.experimental import pallas as pl
from jax.experimental.pallas import tpu as pltpu
from jax.experimental.pallas import tpu_sc as plsc

BATCH = 16384
DIM = 64
NUM_CORES = 2
NUM_SUBCORES = 16
NUM_WORKERS = NUM_CORES * NUM_SUBCORES
ROWS_PER_WORKER = BATCH // NUM_WORKERS
IDX_CHUNK = 128
NUM_CHUNKS = ROWS_PER_WORKER // IDX_CHUNK


def _sc_body(x_hbm, classes_hbm, biases_hbm, out_hbm, idx_v, acc_v, sem):
    wid = lax.axis_index("s") * NUM_CORES + lax.axis_index("c")
    base = wid * ROWS_PER_WORKER
    pltpu.sync_copy(classes_hbm.at[wid], idx_v)
    pltpu.sync_copy(x_hbm.at[pl.ds(base, ROWS_PER_WORKER)], acc_v)
    descs = [
        pltpu.async_copy(
            biases_hbm.at[idx_v.at[j]],
            acc_v.at[pl.ds(j * IDX_CHUNK, IDX_CHUNK)],
            sem,
            add=True,
        )
        for j in range(NUM_CHUNKS)
    ]
    for d in descs:
        d.wait()
    pltpu.sync_copy(acc_v, out_hbm.at[pl.ds(base, ROWS_PER_WORKER)])


@jax.jit
def kernel(x, classes, biases):
    classes_r = classes.astype(jnp.int32).reshape(
        NUM_WORKERS, NUM_CHUNKS, IDX_CHUNK
    )
    mesh = plsc.VectorSubcoreMesh(
        core_axis_name="c", subcore_axis_name="s"
    )
    run = pl.kernel(
        _sc_body,
        out_type=jax.ShapeDtypeStruct((BATCH, DIM), jnp.float32),
        mesh=mesh,
        scratch_types=[
            pltpu.VMEM((NUM_CHUNKS, IDX_CHUNK), jnp.int32),
            pltpu.VMEM((ROWS_PER_WORKER, DIM), jnp.float32),
            pltpu.SemaphoreType.DMA,
        ],
        compiler_params=pltpu.CompilerParams(use_tc_tiling_on_sc=False),
    )
    return run(x, classes_r, biases)


# speedup vs baseline: 2.6143x; 2.6143x over previous
import jax
import jax.numpy as jnp
from jax import lax
from jax.experimental import pallas as pl
from jax.experimental.pallas import tpu as pltpu
from jax.experimental.pallas import tpu_sc as plsc

BATCH = 16384
DIM = 64
N_CLASSES = 1000000
NUM_CORES = 2
NUM_SUBCORES = 16
NUM_WORKERS = NUM_CORES * NUM_SUBCORES
COLS_PER_WORKER = BATCH // NUM_WORKERS
LANES = 16
LANE_TILE = 128
RING = 4


def _sc_body(xt_hbm, cls_hbm, pt_hbm, out_hbm, cls_v, acc, bufs, sems):
    wid = lax.axis_index("s") * NUM_CORES + lax.axis_index("c")
    base = pl.multiple_of(wid * COLS_PER_WORKER, COLS_PER_WORKER)
    pltpu.sync_copy(cls_hbm.at[wid], cls_v.at[:, pl.ds(0, COLS_PER_WORKER)])
    pltpu.sync_copy(xt_hbm.at[:, pl.ds(base, COLS_PER_WORKER)], acc)

    def fire(c, k):
        z = cls_v[0, pl.ds(c, LANES)][0]
        z128 = pl.multiple_of(
            lax.shift_left(lax.shift_right_logical(z, 7), 7), LANE_TILE
        )
        pltpu.async_copy(
            pt_hbm.at[:, pl.ds(z128, LANE_TILE)], bufs.at[k], sems.at[k]
        )

    def drain(k):
        pltpu.make_async_copy(
            pt_hbm.at[:, pl.ds(0, LANE_TILE)], bufs.at[k], sems.at[k]
        ).wait()

    for k in range(RING):
        fire(k, k)

    def col_body(c, carry):
        k = c & (RING - 1)
        drain(k)
        l = cls_v[0, pl.ds(c, LANES)][0] & (LANE_TILE - 1)
        lvec = jnp.full((LANES,), l, jnp.int32)
        kvec = jnp.full((LANES,), k, jnp.int32)
        cvec = jnp.full((LANES,), c, jnp.int32)
        for j in range(DIM // LANES):
            dvec = lax.iota(jnp.int32, LANES) + j * LANES
            v = plsc.load_gather(bufs, [kvec, dvec, lvec])
            plsc.addupdate_scatter(acc, [dvec, cvec], v)

        @pl.when(c + RING < COLS_PER_WORKER)
        def _():
            fire(c + RING, k)

        return carry

    lax.fori_loop(0, COLS_PER_WORKER, col_body, 0)
    pltpu.sync_copy(acc, out_hbm.at[:, pl.ds(base, COLS_PER_WORKER)])


@jax.jit
def kernel(x, classes, biases):
    cls_r = classes.astype(jnp.int32).reshape(NUM_WORKERS, 1, COLS_PER_WORKER)
    mesh = plsc.VectorSubcoreMesh(core_axis_name="c", subcore_axis_name="s")
    run = pl.kernel(
        _sc_body,
        out_type=jax.ShapeDtypeStruct((DIM, BATCH), jnp.float32),
        mesh=mesh,
        scratch_types=[
            pltpu.VMEM((1, COLS_PER_WORKER + LANES), jnp.int32),
            pltpu.VMEM((DIM, COLS_PER_WORKER), jnp.float32),
            pltpu.VMEM((RING, DIM, LANE_TILE), jnp.float32),
            pltpu.SemaphoreType.DMA((RING,)),
        ],
        compiler_params=pltpu.CompilerParams(needs_layout_passes=False),
    )
    out_t = run(x.T, cls_r, biases.T)
    return out_t.T


# speedup vs baseline: 3.0353x; 1.1610x over previous
import jax
import jax.numpy as jnp
from jax import lax
from jax.experimental import pallas as pl
from jax.experimental.pallas import tpu as pltpu
from jax.experimental.pallas import tpu_sc as plsc

BATCH = 16384
DIM = 64
N_CLASSES = 1000000
NUM_CORES = 2
NUM_SUBCORES = 16
NUM_WORKERS = NUM_CORES * NUM_SUBCORES
COLS_PER_WORKER = BATCH // NUM_WORKERS
LANES = 16
LANE_TILE = 128
RING = 8


def _sc_body(xt_hbm, cls_hbm, pt_hbm, out_hbm, cls_v, acc, bufs, sems):
    wid = lax.axis_index("s") * NUM_CORES + lax.axis_index("c")
    base = pl.multiple_of(wid * COLS_PER_WORKER, COLS_PER_WORKER)
    pltpu.sync_copy(cls_hbm.at[wid], cls_v.at[:, pl.ds(0, COLS_PER_WORKER)])
    pltpu.sync_copy(xt_hbm.at[:, pl.ds(base, COLS_PER_WORKER)], acc)

    def fire(c, k):
        z = cls_v[0, pl.ds(c, LANES)][0]
        z128 = pl.multiple_of(
            lax.shift_left(lax.shift_right_logical(z, 7), 7), LANE_TILE
        )
        pltpu.async_copy(
            pt_hbm.at[:, pl.ds(z128, LANE_TILE)], bufs.at[k], sems.at[k]
        )

    def drain(k):
        pltpu.make_async_copy(
            pt_hbm.at[:, pl.ds(0, LANE_TILE)], bufs.at[k], sems.at[k]
        ).wait()

    for k in range(RING):
        fire(k, k)

    def col_body(c, carry):
        k = c & (RING - 1)
        drain(k)
        l = cls_v[0, pl.ds(c, LANES)][0] & (LANE_TILE - 1)
        lvec = jnp.full((LANES,), l, jnp.int32)
        kvec = jnp.full((LANES,), k, jnp.int32)
        cvec = jnp.full((LANES,), c, jnp.int32)
        for j in range(DIM // LANES):
            dvec = lax.iota(jnp.int32, LANES) + j * LANES
            v = plsc.load_gather(bufs, [kvec, dvec, lvec])
            plsc.addupdate_scatter(acc, [dvec, cvec], v)

        @pl.when(c + RING < COLS_PER_WORKER)
        def _():
            fire(c + RING, k)

        return carry

    lax.fori_loop(0, COLS_PER_WORKER, col_body, 0)
    pltpu.sync_copy(acc, out_hbm.at[:, pl.ds(base, COLS_PER_WORKER)])


@jax.jit
def kernel(x, classes, biases):
    cls_r = classes.astype(jnp.int32).reshape(NUM_WORKERS, 1, COLS_PER_WORKER)
    mesh = plsc.VectorSubcoreMesh(core_axis_name="c", subcore_axis_name="s")
    run = pl.kernel(
        _sc_body,
        out_type=jax.ShapeDtypeStruct((DIM, BATCH), jnp.float32),
        mesh=mesh,
        scratch_types=[
            pltpu.VMEM((1, COLS_PER_WORKER + LANES), jnp.int32),
            pltpu.VMEM((DIM, COLS_PER_WORKER), jnp.float32),
            pltpu.VMEM((RING, DIM, LANE_TILE), jnp.float32),
            pltpu.SemaphoreType.DMA((RING,)),
        ],
        compiler_params=pltpu.CompilerParams(needs_layout_passes=False),
    )
    out_t = run(x.T, cls_r, biases.T)
    return out_t.T


# speedup vs baseline: 3.0843x; 1.0162x over previous
import jax
import jax.numpy as jnp
from jax import lax
from jax.experimental import pallas as pl
from jax.experimental.pallas import tpu as pltpu
from jax.experimental.pallas import tpu_sc as plsc

BATCH = 16384
DIM = 64
N_CLASSES = 1000000
NUM_CORES = 2
NUM_SUBCORES = 16
NUM_WORKERS = NUM_CORES * NUM_SUBCORES
COLS_PER_WORKER = BATCH // NUM_WORKERS
LANES = 16
LANE_TILE = 128
RING = 10


def _sc_body(xt_hbm, cls_hbm, pt_hbm, out_hbm, cls_v, acc, bufs, sems):
    wid = lax.axis_index("s") * NUM_CORES + lax.axis_index("c")
    base = pl.multiple_of(wid * COLS_PER_WORKER, COLS_PER_WORKER)
    pltpu.sync_copy(cls_hbm.at[wid], cls_v.at[:, pl.ds(0, COLS_PER_WORKER)])
    pltpu.sync_copy(xt_hbm.at[:, pl.ds(base, COLS_PER_WORKER)], acc)

    def fire(c, k):
        z = cls_v[0, pl.ds(c, LANES)][0]
        z128 = pl.multiple_of(
            lax.shift_left(lax.shift_right_logical(z, 7), 7), LANE_TILE
        )
        pltpu.async_copy(
            pt_hbm.at[:, pl.ds(z128, LANE_TILE)], bufs.at[k], sems.at[k]
        )

    def drain(k):
        pltpu.make_async_copy(
            pt_hbm.at[:, pl.ds(0, LANE_TILE)], bufs.at[k], sems.at[k]
        ).wait()

    for k in range(RING):
        fire(k, k)

    def col_body(c, carry):
        k = lax.rem(c, RING)
        drain(k)
        l = cls_v[0, pl.ds(c, LANES)][0] & (LANE_TILE - 1)
        lvec = jnp.full((LANES,), l, jnp.int32)
        kvec = jnp.full((LANES,), k, jnp.int32)
        cvec = jnp.full((LANES,), c, jnp.int32)
        for j in range(DIM // LANES):
            dvec = lax.iota(jnp.int32, LANES) + j * LANES
            v = plsc.load_gather(bufs, [kvec, dvec, lvec])
            plsc.addupdate_scatter(acc, [dvec, cvec], v)

        @pl.when(c + RING < COLS_PER_WORKER)
        def _():
            fire(c + RING, k)

        return carry

    lax.fori_loop(0, COLS_PER_WORKER, col_body, 0)
    pltpu.sync_copy(acc, out_hbm.at[:, pl.ds(base, COLS_PER_WORKER)])


@jax.jit
def kernel(x, classes, biases):
    cls_r = classes.astype(jnp.int32).reshape(NUM_WORKERS, 1, COLS_PER_WORKER)
    mesh = plsc.VectorSubcoreMesh(core_axis_name="c", subcore_axis_name="s")
    run = pl.kernel(
        _sc_body,
        out_type=jax.ShapeDtypeStruct((DIM, BATCH), jnp.float32),
        mesh=mesh,
        scratch_types=[
            pltpu.VMEM((1, COLS_PER_WORKER + LANES), jnp.int32),
            pltpu.VMEM((DIM, COLS_PER_WORKER), jnp.float32),
            pltpu.VMEM((RING, DIM, LANE_TILE), jnp.float32),
            pltpu.SemaphoreType.DMA((RING,)),
        ],
        compiler_params=pltpu.CompilerParams(needs_layout_passes=False),
    )
    out_t = run(x.T, cls_r, biases.T)
    return out_t.T
